# Initial kernel scaffold; baseline (speedup 1.0000x reference)
import jax
import jax.numpy as jnp
from jax.experimental import pallas as pl


def kernel(inputs, W_gate, W_pre):
    raise NotImplementedError("write your pallas kernel here")


# speedup vs baseline: 1.7495x; 1.7495x over previous
import functools

import jax
import jax.numpy as jnp
from jax.experimental import pallas as pl

_TOP_K = 2
_E = 64
_D = 768
_N = 32768
_T = 1024


def _router_body(x_ref, wg_ref, logits_ref, sel_idx_ref, sel_w_ref, ew_ref):
    x = x_ref[...]
    wg = wg_ref[...]
    logits = jax.lax.dot_general(
        x, wg, (((1,), (1,)), ((), ())), preferred_element_type=jnp.float32)
    logits_ref[...] = logits

    m = jnp.max(logits, axis=1, keepdims=True)
    e = jnp.exp(logits - m)
    p = e / jnp.sum(e, axis=1, keepdims=True)

    iota = jax.lax.broadcasted_iota(jnp.int32, p.shape, 1)
    w1 = jnp.max(p, axis=1, keepdims=True)
    idx1 = jnp.min(jnp.where(p == w1, iota, _E), axis=1, keepdims=True)
    p2 = jnp.where(iota == idx1, -1.0, p)
    w2 = jnp.max(p2, axis=1, keepdims=True)
    idx2 = jnp.min(jnp.where(p2 == w2, iota, _E), axis=1, keepdims=True)

    w1m = jnp.where(idx1 == _E - 1, 0.0, w1)
    w2m = jnp.where(idx2 == _E - 1, 0.0, w2)
    s = w1m + w2m
    w1n = w1m / s
    w2n = w2m / s

    sel_idx_ref[...] = jnp.concatenate([idx1, idx2], axis=1)
    sel_w_ref[...] = jnp.concatenate([w1n, w2n], axis=1)

    ew_t = jnp.where(iota == idx1, w1n, 0.0) + jnp.where(iota == idx2, w2n, 0.0)
    ew_ref[...] = ew_t.T


@functools.partial(jax.jit, static_argnames=())
def kernel(inputs, W_gate, W_pre):
    del W_pre
    x = inputs.astype(jnp.float32)
    n_blocks = _N // _T
    logits, sel_idx, sel_w, ew = pl.pallas_call(
        _router_body,
        grid=(n_blocks,),
        in_specs=[
            pl.BlockSpec((_T, _D), lambda i: (i, 0)),
            pl.BlockSpec((_E, _D), lambda i: (0, 0)),
        ],
        out_specs=[
            pl.BlockSpec((_T, _E), lambda i: (i, 0)),
            pl.BlockSpec((_T, _TOP_K), lambda i: (i, 0)),
            pl.BlockSpec((_T, _TOP_K), lambda i: (i, 0)),
            pl.BlockSpec((_E, _T), lambda i: (0, i)),
        ],
        out_shape=[
            jax.ShapeDtypeStruct((_N, _E), jnp.float32),
            jax.ShapeDtypeStruct((_N, _TOP_K), jnp.int32),
            jax.ShapeDtypeStruct((_N, _TOP_K), jnp.float32),
            jax.ShapeDtypeStruct((_E, _N), jnp.float32),
        ],
    )(x, W_gate)
    return (sel_idx, sel_w, ew, logits)


# speedup vs baseline: 3.4707x; 1.9838x over previous
import jax
import jax.numpy as jnp
from jax.experimental import pallas as pl

_TOP_K = 2
_E = 64
_D = 768
_N = 32768
_T = 1024


def _router_body(x_ref, wg_ref, logits_ref, meta_ref, ew_ref):
    x = x_ref[...]
    wg = wg_ref[...]
    logits = jax.lax.dot_general(
        wg, x, (((1,), (1,)), ((), ())), preferred_element_type=jnp.float32)
    logits_ref[...] = logits

    m = jnp.max(logits, axis=0, keepdims=True)
    e = jnp.exp(logits - m)
    p = e / jnp.sum(e, axis=0, keepdims=True)

    eiota = jax.lax.broadcasted_iota(jnp.int32, p.shape, 0).astype(jnp.float32)
    w1 = jnp.max(p, axis=0, keepdims=True)
    idx1 = jnp.min(jnp.where(p == w1, eiota, float(_E)), axis=0, keepdims=True)
    p2 = jnp.where(eiota == idx1, -1.0, p)
    w2 = jnp.max(p2, axis=0, keepdims=True)
    idx2 = jnp.min(jnp.where(p2 == w2, eiota, float(_E)), axis=0, keepdims=True)

    w1m = jnp.where(idx1 == float(_E - 1), 0.0, w1)
    w2m = jnp.where(idx2 == float(_E - 1), 0.0, w2)
    s = w1m + w2m
    w1n = w1m / s
    w2n = w2m / s

    zeros = jnp.zeros((4, idx1.shape[1]), jnp.float32)
    meta_ref[...] = jnp.concatenate([idx1, idx2, w1n, w2n, zeros], axis=0)

    ew_ref[...] = (jnp.where(eiota == idx1, w1n, 0.0)
                   + jnp.where(eiota == idx2, w2n, 0.0))


@jax.jit
def kernel(inputs, W_gate, W_pre):
    del W_pre
    x = inputs.astype(jnp.float32)
    n_blocks = _N // _T
    logits_t, meta, ew = pl.pallas_call(
        _router_body,
        grid=(n_blocks,),
        in_specs=[
            pl.BlockSpec((_T, _D), lambda i: (i, 0)),
            pl.BlockSpec((_E, _D), lambda i: (0, 0)),
        ],
        out_specs=[
            pl.BlockSpec((_E, _T), lambda i: (0, i)),
            pl.BlockSpec((8, _T), lambda i: (0, i)),
            pl.BlockSpec((_E, _T), lambda i: (0, i)),
        ],
        out_shape=[
            jax.ShapeDtypeStruct((_E, _N), jnp.float32),
            jax.ShapeDtypeStruct((8, _N), jnp.float32),
            jax.ShapeDtypeStruct((_E, _N), jnp.float32),
        ],
    )(x, W_gate)
    sel_idx = meta[0:2, :].T.astype(jnp.int32)
    sel_w = meta[2:4, :].T
    return (sel_idx, sel_w, ew, logits_t.T)


# speedup vs baseline: 4.2598x; 1.2274x over previous
import jax
import jax.numpy as jnp
from jax.experimental import pallas as pl

_TOP_K = 2
_E = 64
_D = 768
_N = 32768
_T = 2048


def _router_body(x_ref, wg_ref, logits_ref, meta_ref, ew_ref):
    x = x_ref[...]
    wg = wg_ref[...]
    logits = jax.lax.dot_general(
        wg, x, (((1,), (1,)), ((), ())), preferred_element_type=jnp.float32)
    logits_ref[...] = logits

    m = jnp.max(logits, axis=0, keepdims=True)
    e = jnp.exp(logits - m)
    p = e / jnp.sum(e, axis=0, keepdims=True)

    eiota = jax.lax.broadcasted_iota(jnp.int32, p.shape, 0).astype(jnp.float32)
    w1 = jnp.max(p, axis=0, keepdims=True)
    idx1 = jnp.min(jnp.where(p == w1, eiota, float(_E)), axis=0, keepdims=True)
    p2 = jnp.where(eiota == idx1, -1.0, p)
    w2 = jnp.max(p2, axis=0, keepdims=True)
    idx2 = jnp.min(jnp.where(p2 == w2, eiota, float(_E)), axis=0, keepdims=True)

    w1m = jnp.where(idx1 == float(_E - 1), 0.0, w1)
    w2m = jnp.where(idx2 == float(_E - 1), 0.0, w2)
    s = w1m + w2m
    w1n = w1m / s
    w2n = w2m / s

    zeros = jnp.zeros((4, idx1.shape[1]), jnp.float32)
    meta_ref[...] = jnp.concatenate([idx1, idx2, w1n, w2n, zeros], axis=0)

    ew_ref[...] = (jnp.where(eiota == idx1, w1n, 0.0)
                   + jnp.where(eiota == idx2, w2n, 0.0))


@jax.jit
def kernel(inputs, W_gate, W_pre):
    del W_pre
    x = inputs.astype(jnp.float32)
    n_blocks = _N // _T
    logits_t, meta, ew = pl.pallas_call(
        _router_body,
        grid=(n_blocks,),
        in_specs=[
            pl.BlockSpec((_T, _D), lambda i: (i, 0)),
            pl.BlockSpec((_E, _D), lambda i: (0, 0)),
        ],
        out_specs=[
            pl.BlockSpec((_E, _T), lambda i: (0, i)),
            pl.BlockSpec((8, _T), lambda i: (0, i)),
            pl.BlockSpec((_E, _T), lambda i: (0, i)),
        ],
        out_shape=[
            jax.ShapeDtypeStruct((_E, _N), jnp.float32),
            jax.ShapeDtypeStruct((8, _N), jnp.float32),
            jax.ShapeDtypeStruct((_E, _N), jnp.float32),
        ],
    )(x, W_gate)
    sel_idx = meta[0:2, :].T.astype(jnp.int32)
    sel_w = meta[2:4, :].T
    return (sel_idx, sel_w, ew, logits_t.T)


# speedup vs baseline: 4.4261x; 1.0390x over previous
import jax
import jax.numpy as jnp
from jax.experimental import pallas as pl

_TOP_K = 2
_E = 64
_D = 768
_N = 32768
_T = 4096


def _router_body(x_ref, wg_ref, logits_ref, meta_ref, ew_ref):
    x = x_ref[...]
    wg = wg_ref[...]
    logits = jax.lax.dot_general(
        wg, x, (((1,), (1,)), ((), ())), preferred_element_type=jnp.float32)
    logits_ref[...] = logits

    m = jnp.max(logits, axis=0, keepdims=True)
    e = jnp.exp(logits - m)
    p = e / jnp.sum(e, axis=0, keepdims=True)

    eiota = jax.lax.broadcasted_iota(jnp.int32, p.shape, 0).astype(jnp.float32)
    w1 = jnp.max(p, axis=0, keepdims=True)
    idx1 = jnp.min(jnp.where(p == w1, eiota, float(_E)), axis=0, keepdims=True)
    p2 = jnp.where(eiota == idx1, -1.0, p)
    w2 = jnp.max(p2, axis=0, keepdims=True)
    idx2 = jnp.min(jnp.where(p2 == w2, eiota, float(_E)), axis=0, keepdims=True)

    w1m = jnp.where(idx1 == float(_E - 1), 0.0, w1)
    w2m = jnp.where(idx2 == float(_E - 1), 0.0, w2)
    s = w1m + w2m
    w1n = w1m / s
    w2n = w2m / s

    zeros = jnp.zeros((4, idx1.shape[1]), jnp.float32)
    meta_ref[...] = jnp.concatenate([idx1, idx2, w1n, w2n, zeros], axis=0)

    ew_ref[...] = (jnp.where(eiota == idx1, w1n, 0.0)
                   + jnp.where(eiota == idx2, w2n, 0.0))


@jax.jit
def kernel(inputs, W_gate, W_pre):
    del W_pre
    x = inputs.astype(jnp.float32)
    n_blocks = _N // _T
    logits_t, meta, ew = pl.pallas_call(
        _router_body,
        grid=(n_blocks,),
        in_specs=[
            pl.BlockSpec((_T, _D), lambda i: (i, 0)),
            pl.BlockSpec((_E, _D), lambda i: (0, 0)),
        ],
        out_specs=[
            pl.BlockSpec((_E, _T), lambda i: (0, i)),
            pl.BlockSpec((8, _T), lambda i: (0, i)),
            pl.BlockSpec((_E, _T), lambda i: (0, i)),
        ],
        out_shape=[
            jax.ShapeDtypeStruct((_E, _N), jnp.float32),
            jax.ShapeDtypeStruct((8, _N), jnp.float32),
            jax.ShapeDtypeStruct((_E, _N), jnp.float32),
        ],
    )(x, W_gate)
    sel_idx = meta[0:2, :].T.astype(jnp.int32)
    sel_w = meta[2:4, :].T
    return (sel_idx, sel_w, ew, logits_t.T)
